# baseline (speedup 1.0000x reference)
import functools
import jax
import jax.numpy as jnp
from jax import lax
from jax.experimental import pallas as pl
from jax.experimental.pallas import tpu as pltpu
from jax.experimental.pallas import tpu_sc as plsc

N_NODES = 50000
N_EDGES = 800000
D_NODE = 128
D_EDGE = 16
B = 1024
K = 20
EPS = 1e-6
NEG = -3.4e38

NC = 2
NS = 16
NW = NC * NS

PAIR_N = B * K + B * K * K
NROUND = 2
PER_TILE = PAIR_N // NS // NROUND
PCHUNK = 2688
NVREG = PCHUNK // 16
SEG = N_EDGES // NW
LOCAL_SZ = PER_TILE + NS * 256


def _bucket_of(idv, inv_seg, seg):
  g = (idv.astype(jnp.float32) * inv_seg).astype(jnp.int32)
  r = idv - g * seg
  g = jnp.where(r < 0, g - 1, g)
  g = jnp.where(idv - g * seg >= seg, g + 1, g)
  return g


def _scmax_body(ids_hbm, ws_hbm, ew_hbm,
                ids_st, ws_st, hist_v, offs_v, allcnt_v, cnt256_v, cntraw_v, lb_v,
                ids_rt, ws_rt, ew_v, ids_sc, ws_sc, stg_i, stg_w,
                cnt_sp, ids_sp, ws_sp, sem):
  cid = lax.axis_index("c")
  tid = lax.axis_index("s")
  lane = lax.iota(jnp.int32, 16)
  zero_i = jnp.zeros((16,), jnp.int32)
  inv_seg = 1.0 / SEG

  seg_base = (cid * NS + tid) * SEG

  def zseg(i, _):
    ew_v[pl.ds(i * 16, 16)] = jnp.zeros((16,), jnp.float32)
    return _

  lax.fori_loop(0, (SEG + 15) // 16, zseg, None)

  ids_sc[pl.ds(0, 16)] = zero_i - 1
  ids_sc[pl.ds(24, 16)] = zero_i - 1

  for rnd in range(NROUND):
    pair0 = rnd * (PAIR_N // NROUND) + tid * PER_TILE

    for i in range(16):
      hist_v[pl.ds(i * 16, 16)] = zero_i

    cpi = pltpu.async_copy(
        ids_hbm.at[pl.ds(pl.multiple_of(pair0, 8), PER_TILE)], ids_st, sem)
    cpw = pltpu.async_copy(
        ws_hbm.at[pl.ds(pl.multiple_of(pair0, 8), PER_TILE)], ws_st, sem)
    cpi.wait()
    cpw.wait()

    def vr_a(v, _):
      idv = ids_st[pl.ds(v * 16, 16)]
      g = _bucket_of(idv, inv_seg, SEG)
      b = g - cid * NS
      bc = jnp.clip(b, 0, NS - 1)
      validi = jnp.where(b >= 0, 1, 0) * jnp.where(b < NS, 1, 0)
      plsc.addupdate_scatter(hist_v, [bc * 16 + lane], validi)
      return _

    lax.fori_loop(0, PER_TILE // 16, vr_a, None)

    pltpu.sync_copy(hist_v, cnt_sp.at[pl.ds(pl.multiple_of(tid * 256, 8), 256)])
    plsc.subcore_barrier()
    pltpu.sync_copy(cnt_sp, allcnt_v)

    def grp(g, _):
      i = g * 16 + lane
      t = i >> 4
      bb = i & (NS - 1)
      def lsum(l, a):
        return a + plsc.load_gather(allcnt_v, [t * 256 + bb * 16 + l])
      acc = lax.fori_loop(0, 16, lsum, zero_i)
      cnt256_v[pl.ds(g * 16, 16)] = ((acc + 255) >> 8) << 8
      cntraw_v[pl.ds(g * 16, 16)] = acc
      return _

    lax.fori_loop(0, 16, grp, None)

    def lb_t(t, _):
      v = cnt256_v[pl.ds(t * 16, 16)]
      lb_v[pl.ds(t * 16, 16)] = plsc.cumsum(v) - v
      return _

    lax.fori_loop(0, 16, lb_t, None)

    def offs_b(b, _):
      hv = hist_v[pl.ds(b * 16, 16)]
      excl = plsc.cumsum(hv) - hv
      base = lb_v[pl.ds(tid * 16 + b, 16)][0]
      offs_v[pl.ds(b * 16, 16)] = base + excl
      return _

    lax.fori_loop(0, NS, offs_b, None)

    def sent(i, _):
      ids_rt[pl.ds(i * 16, 16)] = zero_i - 1
      return _

    lax.fori_loop(0, LOCAL_SZ // 16, sent, None)

    def vr_b(v, _):
      idv = ids_st[pl.ds(v * 16, 16)]
      wv = ws_st[pl.ds(v * 16, 16)]
      g = _bucket_of(idv, inv_seg, SEG)
      b = g - cid * NS
      bc = jnp.clip(b, 0, NS - 1)
      validi = jnp.where(b >= 0, 1, 0) * jnp.where(b < NS, 1, 0)
      valid = validi > 0
      oidx = bc * 16 + lane
      cur = plsc.load_gather(offs_v, [oidx])
      plsc.store_scatter(ids_rt, [cur], idv, mask=valid)
      plsc.store_scatter(ws_rt, [cur], wv, mask=valid)
      plsc.store_scatter(offs_v, [oidx], cur + validi)
      return _

    lax.fori_loop(0, PER_TILE // 16, vr_b, None)

    pltpu.sync_copy(ids_rt, ids_sp.at[pl.ds(pl.multiple_of(tid * LOCAL_SZ, 8), LOCAL_SZ)])
    pltpu.sync_copy(ws_rt, ws_sp.at[pl.ds(pl.multiple_of(tid * LOCAL_SZ, 8), LOCAL_SZ)])
    plsc.subcore_barrier()

    def prod_t(t, _):
      lb_s = lb_v[pl.ds(t * 16 + tid, 16)][0]
      units = cnt256_v[pl.ds(t * 16 + tid, 16)][0] >> 8
      nvreg_raw = (cntraw_v[pl.ds(t * 16 + tid, 16)][0] + 15) >> 4
      src0 = pl.multiple_of(t * LOCAL_SZ + lb_s, 8)

      def unit_u(u, __):
        ci = pltpu.async_copy(
            ids_sp.at[pl.ds(pl.multiple_of(src0 + u * 256, 8), 256)], stg_i, sem)
        cw = pltpu.async_copy(
            ws_sp.at[pl.ds(pl.multiple_of(src0 + u * 256, 8), 256)], stg_w, sem)
        ci.wait()
        cw.wait()

        def vr(v, ___):
          idv = stg_i[pl.ds(v * 16, 16)]
          wv = stg_w[pl.ds(v * 16, 16)]
          valid = idv >= 0
          idl = jnp.where(valid, idv - seg_base, 0)
          wv = jnp.where(valid, wv, NEG)
          ks, vs = plsc.sort_key_val(idl, wv)
          ids_sc[pl.ds(8, 16)] = ks
          cur = vs
          for k in (1, 2, 4, 8):
            ws_sc[pl.ds(8, 16)] = cur
            idk = ids_sc[pl.ds(8 - k, 16)]
            wk = ws_sc[pl.ds(8 - k, 16)]
            cur = jnp.where(idk == ks, jnp.maximum(cur, wk), cur)
          lastm = ids_sc[pl.ds(9, 16)] != ks
          old = plsc.load_gather(ew_v, [ks], mask=lastm)
          plsc.store_scatter(ew_v, [ks], jnp.maximum(old, cur), mask=lastm)
          return ___

        lax.fori_loop(0, jnp.clip(nvreg_raw - u * 16, 0, 16), vr, None)
        return __

      lax.fori_loop(0, units, unit_u, None)
      return _

    lax.fori_loop(0, NS, prod_t, None)
    plsc.subcore_barrier()

  pltpu.sync_copy(ew_v.at[pl.ds(0, SEG)], ew_hbm.at[pl.ds(pl.multiple_of(seg_base, 8), SEG)])


_scmax = pl.kernel(
    _scmax_body,
    out_type=jax.ShapeDtypeStruct((N_EDGES,), jnp.float32),
    mesh=plsc.VectorSubcoreMesh(core_axis_name="c", subcore_axis_name="s"),
    compiler_params=pltpu.CompilerParams(
        use_tc_tiling_on_sc=False, needs_layout_passes=False),
    scratch_types=[
        pltpu.VMEM((PER_TILE,), jnp.int32),
        pltpu.VMEM((PER_TILE,), jnp.float32),
        pltpu.VMEM((256,), jnp.int32),
        pltpu.VMEM((256,), jnp.int32),
        pltpu.VMEM((16 * 256,), jnp.int32),
        pltpu.VMEM((256 + 16,), jnp.int32),
        pltpu.VMEM((256 + 16,), jnp.int32),
        pltpu.VMEM((256 + 16,), jnp.int32),
        pltpu.VMEM((LOCAL_SZ,), jnp.int32),
        pltpu.VMEM((LOCAL_SZ,), jnp.float32),
        pltpu.VMEM((SEG + 16,), jnp.float32),
        pltpu.VMEM((40,), jnp.int32),
        pltpu.VMEM((40,), jnp.float32),
        pltpu.VMEM((256,), jnp.int32),
        pltpu.VMEM((256,), jnp.float32),
        pltpu.VMEM_SHARED((NS * 256,), jnp.int32),
        pltpu.VMEM_SHARED((NS * LOCAL_SZ + 256,), jnp.int32),
        pltpu.VMEM_SHARED((NS * LOCAL_SZ + 256,), jnp.float32),
        pltpu.SemaphoreType.DMA,
    ],
)

B_PER_W = B // NW
NG = K * K + K + 4
NG_PAD = 512
CHUNKS = ((0, 128), (128, 128), (256, 128), (384, 40))
OUT_W = 384


def _sc_body(nidx_hbm, eidx_hbm, ew_hbm, node_hbm, edge_hbm,
             out_hbm,
             idxn_a, idxe_a, ewg_a, w_a, rows_a, erows_a,
             idxn_b, idxe_b, ewg_b, w_b, rows_b, erows_b,
             out_v, sem_a, sem_b):
  wid = lax.axis_index("s") * NC + lax.axis_index("c")
  zero = jnp.zeros((16,), jnp.float32)

  for z in range(304 // 16, OUT_W // 16):
    out_v[pl.ds(z * 16, 16)] = zero

  bufs = ((idxn_a, idxe_a, ewg_a, w_a, rows_a, erows_a, sem_a),
          (idxn_b, idxe_b, ewg_b, w_b, rows_b, erows_b, sem_b))

  def stage_and_fire(gb, bufset):
    idxn_v, idxe_v, ewg_v, w_v, rows_v, erows_v, sem = bufset
    pltpu.sync_copy(nidx_hbm.at[gb], idxn_v)
    pltpu.sync_copy(eidx_hbm.at[gb], idxe_v)
    for off, ln in CHUNKS:
      pltpu.async_copy(
          node_hbm.at[idxn_v.at[pl.ds(off, ln)]], rows_v.at[pl.ds(off, ln)], sem)
      pltpu.async_copy(
          edge_hbm.at[idxe_v.at[pl.ds(off, ln)]], erows_v.at[pl.ds(off, ln)], sem)
      pltpu.async_copy(
          ew_hbm.at[idxe_v.at[pl.ds(off, ln)]], ewg_v.at[pl.ds(off, ln)], sem)

  def drain(bufset):
    idxn_v, idxe_v, ewg_v, w_v, rows_v, erows_v, sem = bufset
    for off, ln in CHUNKS:
      pltpu.make_async_copy(
          node_hbm.at[idxn_v.at[pl.ds(off, ln)]], rows_v.at[pl.ds(off, ln)], sem).wait()
      pltpu.make_async_copy(
          edge_hbm.at[idxe_v.at[pl.ds(off, ln)]], erows_v.at[pl.ds(off, ln)], sem).wait()
      pltpu.make_async_copy(
          ew_hbm.at[idxe_v.at[pl.ds(off, ln)]], ewg_v.at[pl.ds(off, ln)], sem).wait()

  def compute(gb, bufset):
    idxn_v, idxe_v, ewg_v, w_v, rows_v, erows_v, sem = bufset

    for v in range(NG // 16 + 1):
      idn = idxn_v[pl.ds(v * 16, 16)]
      ewv = ewg_v[pl.ds(v * 16, 16)]
      w_v[pl.ds(v * 16, 16)] = jnp.where(idn > 0, ewv, 0.0)

    def hop2_i(i, pcar):
      def hop2_j(j, jcar):
        r = i * K + j
        w = w_v[pl.ds(r, 16)][0]
        accs = tuple(jcar[v] + w * rows_v[r, pl.ds(v * 16, 16)] for v in range(8))
        acce = jcar[8] + w * erows_v[r, :]
        s2 = jcar[9] + w
        return accs + (acce, s2)

      jcar = lax.fori_loop(0, K, hop2_j, (zero,) * 10)
      w1_i = w_v[pl.ds(K * K + i, 16)][0]
      u2 = (w1_i + zero) / (jcar[9] + EPS)
      pf = tuple(pcar[v] + u2 * jcar[v] for v in range(8))
      pe = pcar[8] + u2 * jcar[8]
      return pf + (pe,)

    pcar = lax.fori_loop(0, K, hop2_i, (zero,) * 9)

    def hop1_i(i, fcar):
      r = K * K + i
      w = w_v[pl.ds(r, 16)][0]
      accs = tuple(fcar[v] + w * rows_v[r, pl.ds(v * 16, 16)] for v in range(8))
      acce = fcar[8] + w * erows_v[r, :]
      s1 = fcar[9] + w
      return accs + (acce, s1)

    fcar = lax.fori_loop(0, K, hop1_i, (zero,) * 10)

    for v in range(8):
      out_v[pl.ds(v * 16, 16)] = pcar[v]
      out_v[pl.ds(128 + v * 16, 16)] = fcar[v]
    out_v[pl.ds(256, 16)] = pcar[8]
    out_v[pl.ds(272, 16)] = fcar[8]
    out_v[pl.ds(288, 16)] = fcar[9]
    pltpu.sync_copy(out_v, out_hbm.at[gb])

  stage_and_fire(wid * B_PER_W, bufs[0])

  def pair(p, _):
    s0 = wid * B_PER_W + 2 * p
    stage_and_fire(s0 + 1, bufs[1])
    drain(bufs[0])
    compute(s0, bufs[0])
    stage_and_fire(jnp.minimum(s0 + 2, B - 1), bufs[0])
    drain(bufs[1])
    compute(s0 + 1, bufs[1])
    return _

  lax.fori_loop(0, B_PER_W // 2, pair, None)
  drain(bufs[0])


_sc_gather = pl.kernel(
    _sc_body,
    out_type=jax.ShapeDtypeStruct((B, OUT_W), jnp.float32),
    mesh=plsc.VectorSubcoreMesh(core_axis_name="c", subcore_axis_name="s"),
    compiler_params=pltpu.CompilerParams(use_tc_tiling_on_sc=False),
    scratch_types=[
        pltpu.VMEM((NG_PAD,), jnp.int32),
        pltpu.VMEM((NG_PAD,), jnp.int32),
        pltpu.VMEM((NG_PAD,), jnp.float32),
        pltpu.VMEM((NG_PAD + 16,), jnp.float32),
        pltpu.VMEM((NG, D_NODE), jnp.float32),
        pltpu.VMEM((NG, D_EDGE), jnp.float32),
        pltpu.VMEM((NG_PAD,), jnp.int32),
        pltpu.VMEM((NG_PAD,), jnp.int32),
        pltpu.VMEM((NG_PAD,), jnp.float32),
        pltpu.VMEM((NG_PAD + 16,), jnp.float32),
        pltpu.VMEM((NG, D_NODE), jnp.float32),
        pltpu.VMEM((NG, D_EDGE), jnp.float32),
        pltpu.VMEM((OUT_W,), jnp.float32),
        pltpu.SemaphoreType.DMA,
        pltpu.SemaphoreType.DMA,
    ],
)


def _tc_body(agg_ref, we_ref, wproj_ref, out_ref):
  a = agg_ref[...]
  pf = a[:, 0:128]
  accf1 = a[:, 128:256]
  pe = a[:, 256:272]
  acce1 = a[:, 272:288]
  s1 = a[:, 288:289]
  we = we_ref[...]
  t = pf + jnp.dot(pe, we, preferred_element_type=jnp.float32)
  out = (accf1 + jnp.dot(acce1, we, preferred_element_type=jnp.float32)
         + jnp.dot(t, wproj_ref[...], preferred_element_type=jnp.float32))
  out_ref[...] = out / (s1 + EPS)


def kernel(node_table, edge_table, W_e, W_proj, edge_imp_0, edge_imp_1,
           hop1_nodes, hop1_eidx, hop2_nodes, hop2_eidx):
  flat_e = jnp.concatenate(
      [hop1_eidx.reshape(-1), hop2_eidx.reshape(-1)]).astype(jnp.int32)
  flat_w = jnp.concatenate([edge_imp_0.reshape(-1), edge_imp_1.reshape(-1)])

  ew = _scmax(flat_e, flat_w)

  npad = NG_PAD - (K * K + K)
  pad_n = (jnp.arange(B, dtype=jnp.int32)[:, None] * npad
           + jnp.arange(npad, dtype=jnp.int32)[None, :]) % N_NODES
  pad_e = (jnp.arange(B, dtype=jnp.int32)[:, None] * npad
           + jnp.arange(npad, dtype=jnp.int32)[None, :]) % N_EDGES
  nidx = jnp.concatenate(
      [hop2_nodes.reshape(B, K * K), hop1_nodes, pad_n], axis=1).astype(jnp.int32)
  eidx = jnp.concatenate(
      [hop2_eidx.reshape(B, K * K), hop1_eidx, pad_e], axis=1).astype(jnp.int32)

  agg = _sc_gather(nidx, eidx, ew, node_table, edge_table)

  return pl.pallas_call(
      _tc_body,
      out_shape=jax.ShapeDtypeStruct((B, D_NODE), jnp.float32),
  )(agg, W_e, W_proj)


# speedup vs baseline: 1.0298x; 1.0298x over previous
import functools
import jax
import jax.numpy as jnp
from jax import lax
from jax.experimental import pallas as pl
from jax.experimental.pallas import tpu as pltpu
from jax.experimental.pallas import tpu_sc as plsc

N_NODES = 50000
N_EDGES = 800000
D_NODE = 128
D_EDGE = 16
B = 1024
K = 20
EPS = 1e-6
NEG = -3.4e38

NC = 2
NS = 16
NW = NC * NS

PAIR_N = B * K + B * K * K
NROUND = 2
PER_TILE = PAIR_N // NS // NROUND
PCHUNK = 2688
NVREG = PCHUNK // 16
SEG = N_EDGES // NW
LOCAL_SZ = PER_TILE + NS * 256


def _bucket_of(idv, inv_seg, seg):
  g = (idv.astype(jnp.float32) * inv_seg).astype(jnp.int32)
  r = idv - g * seg
  g = jnp.where(r < 0, g - 1, g)
  g = jnp.where(idv - g * seg >= seg, g + 1, g)
  return g


def _scmax_body(ids_hbm, ws_hbm, ew_hbm,
                ids_st, ws_st, hist_v, offs_v, allcnt_v, cnt256_v, cntraw_v, lb_v,
                ids_rt, ws_rt, ew_v, ids_sc, ws_sc, stg_i, stg_w,
                cnt_sp, ids_sp, ws_sp, sem):
  cid = lax.axis_index("c")
  tid = lax.axis_index("s")
  lane = lax.iota(jnp.int32, 16)
  zero_i = jnp.zeros((16,), jnp.int32)
  inv_seg = 1.0 / SEG

  seg_base = (cid * NS + tid) * SEG

  def zseg(i, _):
    ew_v[pl.ds(i * 16, 16)] = jnp.zeros((16,), jnp.float32)
    return _

  lax.fori_loop(0, (SEG + 15) // 16, zseg, None)

  ids_sc[pl.ds(0, 16)] = zero_i - 1
  ids_sc[pl.ds(24, 16)] = zero_i - 1

  for rnd in range(NROUND):
    pair0 = rnd * (PAIR_N // NROUND) + tid * PER_TILE

    for i in range(16):
      hist_v[pl.ds(i * 16, 16)] = zero_i

    cpi = pltpu.async_copy(
        ids_hbm.at[pl.ds(pl.multiple_of(pair0, 8), PER_TILE)], ids_st, sem)
    cpw = pltpu.async_copy(
        ws_hbm.at[pl.ds(pl.multiple_of(pair0, 8), PER_TILE)], ws_st, sem)
    cpi.wait()
    cpw.wait()

    def vr_a(v, _):
      idv = ids_st[pl.ds(v * 16, 16)]
      g = _bucket_of(idv, inv_seg, SEG)
      b = g - cid * NS
      bc = jnp.clip(b, 0, NS - 1)
      validi = jnp.where(b >= 0, 1, 0) * jnp.where(b < NS, 1, 0)
      plsc.addupdate_scatter(hist_v, [bc * 16 + lane], validi)
      return _

    lax.fori_loop(0, PER_TILE // 16, vr_a, None)

    pltpu.sync_copy(hist_v, cnt_sp.at[pl.ds(pl.multiple_of(tid * 256, 8), 256)])
    plsc.subcore_barrier()
    pltpu.sync_copy(cnt_sp, allcnt_v)

    def grp(g, _):
      i = g * 16 + lane
      t = i >> 4
      bb = i & (NS - 1)
      def lsum(l, a):
        return a + plsc.load_gather(allcnt_v, [t * 256 + bb * 16 + l])
      acc = lax.fori_loop(0, 16, lsum, zero_i)
      cnt256_v[pl.ds(g * 16, 16)] = ((acc + 255) >> 8) << 8
      cntraw_v[pl.ds(g * 16, 16)] = acc
      return _

    lax.fori_loop(0, 16, grp, None)

    def lb_t(t, _):
      v = cnt256_v[pl.ds(t * 16, 16)]
      lb_v[pl.ds(t * 16, 16)] = plsc.cumsum(v) - v
      return _

    lax.fori_loop(0, 16, lb_t, None)

    def offs_b(b, _):
      hv = hist_v[pl.ds(b * 16, 16)]
      excl = plsc.cumsum(hv) - hv
      base = lb_v[pl.ds(tid * 16 + b, 16)][0]
      offs_v[pl.ds(b * 16, 16)] = base + excl
      return _

    lax.fori_loop(0, NS, offs_b, None)

    def sent(i, _):
      ids_rt[pl.ds(i * 16, 16)] = zero_i - 1
      return _

    lax.fori_loop(0, LOCAL_SZ // 16, sent, None)

    def vr_b(v, _):
      idv = ids_st[pl.ds(v * 16, 16)]
      wv = ws_st[pl.ds(v * 16, 16)]
      g = _bucket_of(idv, inv_seg, SEG)
      b = g - cid * NS
      bc = jnp.clip(b, 0, NS - 1)
      validi = jnp.where(b >= 0, 1, 0) * jnp.where(b < NS, 1, 0)
      valid = validi > 0
      oidx = bc * 16 + lane
      cur = plsc.load_gather(offs_v, [oidx])
      plsc.store_scatter(ids_rt, [cur], idv, mask=valid)
      plsc.store_scatter(ws_rt, [cur], wv, mask=valid)
      plsc.store_scatter(offs_v, [oidx], cur + validi)
      return _

    lax.fori_loop(0, PER_TILE // 16, vr_b, None)

    pltpu.sync_copy(ids_rt, ids_sp.at[pl.ds(pl.multiple_of(tid * LOCAL_SZ, 8), LOCAL_SZ)])
    pltpu.sync_copy(ws_rt, ws_sp.at[pl.ds(pl.multiple_of(tid * LOCAL_SZ, 8), LOCAL_SZ)])
    plsc.subcore_barrier()

    def prod_t(t, _):
      lb_s = lb_v[pl.ds(t * 16 + tid, 16)][0]
      units = cnt256_v[pl.ds(t * 16 + tid, 16)][0] >> 8
      nvreg_raw = (cntraw_v[pl.ds(t * 16 + tid, 16)][0] + 15) >> 4
      src0 = pl.multiple_of(t * LOCAL_SZ + lb_s, 8)

      def unit_u(u, __):
        ci = pltpu.async_copy(
            ids_sp.at[pl.ds(pl.multiple_of(src0 + u * 256, 8), 256)], stg_i, sem)
        cw = pltpu.async_copy(
            ws_sp.at[pl.ds(pl.multiple_of(src0 + u * 256, 8), 256)], stg_w, sem)
        ci.wait()
        cw.wait()

        def vr(v, ___):
          idv = stg_i[pl.ds(v * 16, 16)]
          wv = stg_w[pl.ds(v * 16, 16)]
          valid = idv >= 0
          idl = jnp.where(valid, idv - seg_base, 0)
          wv = jnp.where(valid, wv, NEG)
          ks, vs = plsc.sort_key_val(idl, wv)
          ids_sc[pl.ds(8, 16)] = ks
          cur = vs
          for k in (1, 2, 4, 8):
            ws_sc[pl.ds(8, 16)] = cur
            idk = ids_sc[pl.ds(8 - k, 16)]
            wk = ws_sc[pl.ds(8 - k, 16)]
            cur = jnp.where(idk == ks, jnp.maximum(cur, wk), cur)
          lastm = ids_sc[pl.ds(9, 16)] != ks
          old = plsc.load_gather(ew_v, [ks], mask=lastm)
          plsc.store_scatter(ew_v, [ks], jnp.maximum(old, cur), mask=lastm)
          return ___

        lax.fori_loop(0, jnp.clip(nvreg_raw - u * 16, 0, 16), vr, None)
        return __

      lax.fori_loop(0, units, unit_u, None)
      return _

    lax.fori_loop(0, NS, prod_t, None)
    plsc.subcore_barrier()

  pltpu.sync_copy(ew_v.at[pl.ds(0, SEG)], ew_hbm.at[pl.ds(pl.multiple_of(seg_base, 8), SEG)])


_scmax = pl.kernel(
    _scmax_body,
    out_type=jax.ShapeDtypeStruct((N_EDGES,), jnp.float32),
    mesh=plsc.VectorSubcoreMesh(core_axis_name="c", subcore_axis_name="s"),
    compiler_params=pltpu.CompilerParams(
        use_tc_tiling_on_sc=False, needs_layout_passes=False),
    scratch_types=[
        pltpu.VMEM((PER_TILE,), jnp.int32),
        pltpu.VMEM((PER_TILE,), jnp.float32),
        pltpu.VMEM((256,), jnp.int32),
        pltpu.VMEM((256,), jnp.int32),
        pltpu.VMEM((16 * 256,), jnp.int32),
        pltpu.VMEM((256 + 16,), jnp.int32),
        pltpu.VMEM((256 + 16,), jnp.int32),
        pltpu.VMEM((256 + 16,), jnp.int32),
        pltpu.VMEM((LOCAL_SZ,), jnp.int32),
        pltpu.VMEM((LOCAL_SZ,), jnp.float32),
        pltpu.VMEM((SEG + 16,), jnp.float32),
        pltpu.VMEM((40,), jnp.int32),
        pltpu.VMEM((40,), jnp.float32),
        pltpu.VMEM((256,), jnp.int32),
        pltpu.VMEM((256,), jnp.float32),
        pltpu.VMEM_SHARED((NS * 256,), jnp.int32),
        pltpu.VMEM_SHARED((NS * LOCAL_SZ + 256,), jnp.int32),
        pltpu.VMEM_SHARED((NS * LOCAL_SZ + 256,), jnp.float32),
        pltpu.SemaphoreType.DMA,
    ],
)

B_PER_W = B // NW
NG = K * K + K + 4
NG_PAD = 512
CHUNKS = ((0, 128), (128, 128), (256, 128), (384, 40))
OUT_W = 384


def _sc_body(nidx_hbm, eidx_hbm, ew_hbm, node_hbm, edge_hbm,
             out_hbm,
             idxn_a, idxe_a, ewg_a, w_a, rows_a, erows_a,
             idxn_b, idxe_b, ewg_b, w_b, rows_b, erows_b,
             out_v, sem_a, sem_b):
  wid = lax.axis_index("s") * NC + lax.axis_index("c")
  zero = jnp.zeros((16,), jnp.float32)

  for z in range(304 // 16, OUT_W // 16):
    out_v[pl.ds(z * 16, 16)] = zero

  bufs = ((idxn_a, idxe_a, ewg_a, w_a, rows_a, erows_a, sem_a),
          (idxn_b, idxe_b, ewg_b, w_b, rows_b, erows_b, sem_b))

  def stage_and_fire(gb, bufset):
    idxn_v, idxe_v, ewg_v, w_v, rows_v, erows_v, sem = bufset
    ci = pltpu.async_copy(nidx_hbm.at[gb], idxn_v, sem)
    ce = pltpu.async_copy(eidx_hbm.at[gb], idxe_v, sem)
    ci.wait()
    ce.wait()
    for off, ln in CHUNKS:
      pltpu.async_copy(
          node_hbm.at[idxn_v.at[pl.ds(off, ln)]], rows_v.at[pl.ds(off, ln)], sem)
      pltpu.async_copy(
          edge_hbm.at[idxe_v.at[pl.ds(off, ln)]], erows_v.at[pl.ds(off, ln)], sem)
      pltpu.async_copy(
          ew_hbm.at[idxe_v.at[pl.ds(off, ln)]], ewg_v.at[pl.ds(off, ln)], sem)

  def drain(bufset):
    idxn_v, idxe_v, ewg_v, w_v, rows_v, erows_v, sem = bufset
    for off, ln in CHUNKS:
      pltpu.make_async_copy(
          node_hbm.at[idxn_v.at[pl.ds(off, ln)]], rows_v.at[pl.ds(off, ln)], sem).wait()
      pltpu.make_async_copy(
          edge_hbm.at[idxe_v.at[pl.ds(off, ln)]], erows_v.at[pl.ds(off, ln)], sem).wait()
      pltpu.make_async_copy(
          ew_hbm.at[idxe_v.at[pl.ds(off, ln)]], ewg_v.at[pl.ds(off, ln)], sem).wait()

  def compute(gb, bufset):
    idxn_v, idxe_v, ewg_v, w_v, rows_v, erows_v, sem = bufset

    for v in range(NG // 16 + 1):
      idn = idxn_v[pl.ds(v * 16, 16)]
      ewv = ewg_v[pl.ds(v * 16, 16)]
      w_v[pl.ds(v * 16, 16)] = jnp.where(idn > 0, ewv, 0.0)

    def hop2_i(i, pcar):
      def hop2_j(j, jcar):
        r = i * K + j
        w = w_v[pl.ds(r, 16)][0]
        accs = tuple(jcar[v] + w * rows_v[r, pl.ds(v * 16, 16)] for v in range(8))
        acce = jcar[8] + w * erows_v[r, :]
        s2 = jcar[9] + w
        return accs + (acce, s2)

      jcar = lax.fori_loop(0, K, hop2_j, (zero,) * 10)
      w1_i = w_v[pl.ds(K * K + i, 16)][0]
      u2 = (w1_i + zero) / (jcar[9] + EPS)
      pf = tuple(pcar[v] + u2 * jcar[v] for v in range(8))
      pe = pcar[8] + u2 * jcar[8]
      return pf + (pe,)

    pcar = lax.fori_loop(0, K, hop2_i, (zero,) * 9)

    def hop1_i(i, fcar):
      r = K * K + i
      w = w_v[pl.ds(r, 16)][0]
      accs = tuple(fcar[v] + w * rows_v[r, pl.ds(v * 16, 16)] for v in range(8))
      acce = fcar[8] + w * erows_v[r, :]
      s1 = fcar[9] + w
      return accs + (acce, s1)

    fcar = lax.fori_loop(0, K, hop1_i, (zero,) * 10)

    for v in range(8):
      out_v[pl.ds(v * 16, 16)] = pcar[v]
      out_v[pl.ds(128 + v * 16, 16)] = fcar[v]
    out_v[pl.ds(256, 16)] = pcar[8]
    out_v[pl.ds(272, 16)] = fcar[8]
    out_v[pl.ds(288, 16)] = fcar[9]
    pltpu.sync_copy(out_v, out_hbm.at[gb])

  stage_and_fire(wid * B_PER_W, bufs[0])

  def pair(p, _):
    s0 = wid * B_PER_W + 2 * p
    stage_and_fire(s0 + 1, bufs[1])
    drain(bufs[0])
    compute(s0, bufs[0])
    stage_and_fire(jnp.minimum(s0 + 2, B - 1), bufs[0])
    drain(bufs[1])
    compute(s0 + 1, bufs[1])
    return _

  lax.fori_loop(0, B_PER_W // 2, pair, None)
  drain(bufs[0])


_sc_gather = pl.kernel(
    _sc_body,
    out_type=jax.ShapeDtypeStruct((B, OUT_W), jnp.float32),
    mesh=plsc.VectorSubcoreMesh(core_axis_name="c", subcore_axis_name="s"),
    compiler_params=pltpu.CompilerParams(use_tc_tiling_on_sc=False),
    scratch_types=[
        pltpu.VMEM((NG_PAD,), jnp.int32),
        pltpu.VMEM((NG_PAD,), jnp.int32),
        pltpu.VMEM((NG_PAD,), jnp.float32),
        pltpu.VMEM((NG_PAD + 16,), jnp.float32),
        pltpu.VMEM((NG, D_NODE), jnp.float32),
        pltpu.VMEM((NG, D_EDGE), jnp.float32),
        pltpu.VMEM((NG_PAD,), jnp.int32),
        pltpu.VMEM((NG_PAD,), jnp.int32),
        pltpu.VMEM((NG_PAD,), jnp.float32),
        pltpu.VMEM((NG_PAD + 16,), jnp.float32),
        pltpu.VMEM((NG, D_NODE), jnp.float32),
        pltpu.VMEM((NG, D_EDGE), jnp.float32),
        pltpu.VMEM((OUT_W,), jnp.float32),
        pltpu.SemaphoreType.DMA,
        pltpu.SemaphoreType.DMA,
    ],
)


def _tc_body(agg_ref, we_ref, wproj_ref, out_ref):
  a = agg_ref[...]
  pf = a[:, 0:128]
  accf1 = a[:, 128:256]
  pe = a[:, 256:272]
  acce1 = a[:, 272:288]
  s1 = a[:, 288:289]
  we = we_ref[...]
  t = pf + jnp.dot(pe, we, preferred_element_type=jnp.float32)
  out = (accf1 + jnp.dot(acce1, we, preferred_element_type=jnp.float32)
         + jnp.dot(t, wproj_ref[...], preferred_element_type=jnp.float32))
  out_ref[...] = out / (s1 + EPS)


def kernel(node_table, edge_table, W_e, W_proj, edge_imp_0, edge_imp_1,
           hop1_nodes, hop1_eidx, hop2_nodes, hop2_eidx):
  flat_e = jnp.concatenate(
      [hop1_eidx.reshape(-1), hop2_eidx.reshape(-1)]).astype(jnp.int32)
  flat_w = jnp.concatenate([edge_imp_0.reshape(-1), edge_imp_1.reshape(-1)])

  ew = _scmax(flat_e, flat_w)

  npad = NG_PAD - (K * K + K)
  pad_n = (jnp.arange(B, dtype=jnp.int32)[:, None] * npad
           + jnp.arange(npad, dtype=jnp.int32)[None, :]) % N_NODES
  pad_e = (jnp.arange(B, dtype=jnp.int32)[:, None] * npad
           + jnp.arange(npad, dtype=jnp.int32)[None, :]) % N_EDGES
  nidx = jnp.concatenate(
      [hop2_nodes.reshape(B, K * K), hop1_nodes, pad_n], axis=1).astype(jnp.int32)
  eidx = jnp.concatenate(
      [hop2_eidx.reshape(B, K * K), hop1_eidx, pad_e], axis=1).astype(jnp.int32)

  agg = _sc_gather(nidx, eidx, ew, node_table, edge_table)

  return pl.pallas_call(
      _tc_body,
      out_shape=jax.ShapeDtypeStruct((B, D_NODE), jnp.float32),
  )(agg, W_e, W_proj)
